# Initial kernel scaffold; baseline (speedup 1.0000x reference)
import jax
import jax.numpy as jnp
from jax.experimental import pallas as pl


def kernel(x, edge_index, batch, Wa1, ba1, g1, be1, Wb1, bb1, Wa2, ba2, g2, be2, Wb2, bb2, Wa3, ba3, g3, be3, Wb3, bb3, W4, b4, W5, b5):
    raise NotImplementedError("write your pallas kernel here")


# speedup vs baseline: 2.9856x; 2.9856x over previous
import functools

import jax
import jax.numpy as jnp
from jax import lax
from jax.experimental import pallas as pl
from jax.experimental.pallas import tpu as pltpu
from jax.experimental.pallas import tpu_sc as plsc

N = 10000
E = 320000
D = 128
H = 128
G = 64
BN_EPS = 1e-5

NC = 2
NS = 16
NW = NC * NS
EPW = E // NW
CHUNK = 128
EPW_PAD = 10240
ITERS = EPW_PAD // CHUNK
NP = 10240
RPT = NP // NS

_sc_mesh = plsc.VectorSubcoreMesh(core_axis_name="c", subcore_axis_name="s")


@functools.partial(
    pl.kernel,
    mesh=_sc_mesh,
    out_type=jax.ShapeDtypeStruct((NC, NP, D), jnp.float32),
    scratch_types=[
        pltpu.VMEM((ITERS, CHUNK), jnp.int32),
        pltpu.VMEM((ITERS, CHUNK), jnp.int32),
        pltpu.VMEM((CHUNK, D), jnp.float32),
        pltpu.VMEM_SHARED((NP, D), jnp.float32),
        pltpu.SemaphoreType.DMA,
    ],
)
def _sc_aggregate(x_hbm, src_hbm, dst_hbm, z_hbm, out_hbm,
                  src_v, dst_v, rows_v, agg_s, sem):
    c = lax.axis_index("c")
    s = lax.axis_index("s")
    wid = s * NC + c

    pltpu.sync_copy(z_hbm.at[pl.ds(s * RPT, RPT)],
                    agg_s.at[pl.ds(s * RPT, RPT)])
    pltpu.sync_copy(src_hbm.at[wid], src_v)
    pltpu.sync_copy(dst_hbm.at[wid], dst_v)
    plsc.subcore_barrier()

    def body(j, carry):
        pltpu.async_copy(x_hbm.at[src_v.at[j]], rows_v, sem).wait()
        pltpu.sync_copy(rows_v, agg_s.at[dst_v.at[j]], add=True)
        return carry

    lax.fori_loop(0, ITERS, body, 0)
    plsc.subcore_barrier()

    pltpu.sync_copy(agg_s.at[pl.ds(s * RPT, RPT)],
                    out_hbm.at[c, pl.ds(s * RPT, RPT)])


BLK = 1000
NBLK = N // BLK


def _mlp_pool_body(x_ref, a0_ref, a1_ref, b_ref, Wa_ref, sc_ref, sh_ref,
                   Wb_ref, bb_ref, h_ref, p_ref):
    i = pl.program_id(0)
    hin = x_ref[...] + a0_ref[...] + a1_ref[...]
    t = lax.dot_general(hin, Wa_ref[...], (((1,), (1,)), ((), ())),
                        preferred_element_type=jnp.float32)
    t = jnp.maximum(t * sc_ref[...] + sh_ref[...], 0.0)
    h = lax.dot_general(t, Wb_ref[...], (((1,), (1,)), ((), ())),
                        preferred_element_type=jnp.float32)
    h = jnp.maximum(h + bb_ref[...], 0.0)
    h_ref[...] = h

    @pl.when(i == 0)
    def _():
        p_ref[...] = jnp.zeros_like(p_ref)

    seg = b_ref[0, 0, :][None, :]
    mask = (lax.broadcasted_iota(jnp.int32, (G, BLK), 0) == seg
            ).astype(jnp.float32)
    p_ref[...] += lax.dot_general(mask, h, (((1,), (0,)), ((), ())),
                                  preferred_element_type=jnp.float32)


_mlp_pool = pl.pallas_call(
    _mlp_pool_body,
    grid=(NBLK,),
    in_specs=[
        pl.BlockSpec((BLK, D), lambda i: (i, 0)),
        pl.BlockSpec((BLK, D), lambda i: (i, 0)),
        pl.BlockSpec((BLK, D), lambda i: (i, 0)),
        pl.BlockSpec((1, 1, BLK), lambda i: (i, 0, 0)),
        pl.BlockSpec((H, D), lambda i: (0, 0)),
        pl.BlockSpec((1, H), lambda i: (0, 0)),
        pl.BlockSpec((1, H), lambda i: (0, 0)),
        pl.BlockSpec((H, H), lambda i: (0, 0)),
        pl.BlockSpec((1, H), lambda i: (0, 0)),
    ],
    out_specs=[
        pl.BlockSpec((BLK, H), lambda i: (i, 0)),
        pl.BlockSpec((G, H), lambda i: (0, 0)),
    ],
    out_shape=[
        jax.ShapeDtypeStruct((N, H), jnp.float32),
        jax.ShapeDtypeStruct((G, H), jnp.float32),
    ],
    compiler_params=pltpu.CompilerParams(
        dimension_semantics=("arbitrary",)),
)


def _head_body(p1_ref, p2_ref, p3_ref, W4_ref, b4_ref, W5_ref, b5_ref,
               o_ref, sm_ref):
    p = jnp.concatenate([p1_ref[...], p2_ref[...], p3_ref[...]], axis=1)
    t = lax.dot_general(p, W4_ref[...], (((1,), (1,)), ((), ())),
                        preferred_element_type=jnp.float32)
    t = jnp.maximum(t + b4_ref[...], 0.0)
    o = lax.dot_general(t, W5_ref[...], (((1,), (1,)), ((), ())),
                        preferred_element_type=jnp.float32)
    o = o + b5_ref[...]
    o_ref[...] = o
    m = jnp.max(o, axis=1, keepdims=True)
    e = jnp.exp(o - m)
    sm_ref[...] = e / jnp.sum(e, axis=1, keepdims=True)


_head = pl.pallas_call(
    _head_body,
    out_shape=[
        jax.ShapeDtypeStruct((G, 2), jnp.float32),
        jax.ShapeDtypeStruct((G, 2), jnp.float32),
    ],
)


def kernel(x, edge_index, batch,
           Wa1, ba1, g1, be1, Wb1, bb1,
           Wa2, ba2, g2, be2, Wb2, bb2,
           Wa3, ba3, g3, be3, Wb3, bb3,
           W4, b4, W5, b5):
    pad = EPW_PAD - EPW
    src = jnp.concatenate(
        [edge_index[0].reshape(NW, EPW),
         jnp.zeros((NW, pad), jnp.int32)], axis=1).reshape(NW, ITERS, CHUNK)
    dst = jnp.concatenate(
        [edge_index[1].reshape(NW, EPW),
         jnp.full((NW, pad), N, jnp.int32)], axis=1).reshape(NW, ITERS, CHUNK)
    z = jnp.zeros((NP, D), jnp.float32)
    batch3 = batch.reshape(NBLK, 1, BLK)

    inv = 1.0 / jnp.sqrt(jnp.float32(1.0 + BN_EPS))
    h = x
    pools = []
    for (Wa, ba, g, be, Wb, bb) in (
            (Wa1, ba1, g1, be1, Wb1, bb1),
            (Wa2, ba2, g2, be2, Wb2, bb2),
            (Wa3, ba3, g3, be3, Wb3, bb3)):
        agg = _sc_aggregate(h, src, dst, z)
        a0 = agg[0, :N]
        a1 = agg[1, :N]
        scale = (g * inv).reshape(1, H)
        shift = (ba * g * inv + be).reshape(1, H)
        h, p = _mlp_pool(h, a0, a1, batch3, Wa, scale, shift,
                         Wb, bb.reshape(1, H))
        pools.append(p)

    o, sm = _head(pools[0], pools[1], pools[2], W4, b4.reshape(1, 3 * H),
                  W5, b5.reshape(1, 2))
    return (o, sm)


# speedup vs baseline: 3.2099x; 1.0752x over previous
import functools

import jax
import jax.numpy as jnp
from jax import lax
from jax.experimental import pallas as pl
from jax.experimental.pallas import tpu as pltpu
from jax.experimental.pallas import tpu_sc as plsc

N = 10000
E = 320000
D = 128
H = 128
G = 64
BN_EPS = 1e-5

NC = 2
NS = 16
NW = NC * NS
EPW = E // NW
CHUNK = 128
EPW_PAD = 10240
ITERS = EPW_PAD // CHUNK
HALF = ITERS // 2
NP = 10240
RPT = NP // NS

_sc_mesh = plsc.VectorSubcoreMesh(core_axis_name="c", subcore_axis_name="s")


@functools.partial(
    pl.kernel,
    mesh=_sc_mesh,
    out_type=jax.ShapeDtypeStruct((NC, NP, D), jnp.float32),
    scratch_types=[
        pltpu.VMEM((2, CHUNK), jnp.int32),
        pltpu.VMEM((2, CHUNK), jnp.int32),
        pltpu.VMEM((CHUNK, D), jnp.float32),
        pltpu.VMEM((CHUNK, D), jnp.float32),
        pltpu.VMEM_SHARED((NP, D), jnp.float32),
        pltpu.SemaphoreType.DMA,
        pltpu.SemaphoreType.DMA,
        pltpu.SemaphoreType.DMA,
    ],
)
def _sc_aggregate(x_hbm, ed_hbm, z_hbm, out_hbm,
                  idxA, idxB, rowsA, rowsB, agg_s, sem_ga, sem_gb, sem_i):
    c = lax.axis_index("c")
    s = lax.axis_index("s")
    wid = s * NC + c

    pltpu.sync_copy(z_hbm.at[pl.ds(s * RPT, RPT)],
                    agg_s.at[pl.ds(s * RPT, RPT)])
    plsc.subcore_barrier()

    pltpu.sync_copy(ed_hbm.at[wid, 0], idxA)
    pltpu.sync_copy(ed_hbm.at[wid, 1], idxB)
    pltpu.async_copy(x_hbm.at[idxA.at[0]], rowsA, sem_ga)

    def body(jj, carry):
        not_last = jj < HALF - 1
        pltpu.async_copy(x_hbm.at[idxB.at[0]], rowsB, sem_gb)
        pltpu.make_async_copy(x_hbm.at[idxA.at[0]], rowsA, sem_ga).wait()
        pltpu.sync_copy(rowsA, agg_s.at[idxA.at[1]], add=True)

        @pl.when(not_last)
        def _():
            pltpu.async_copy(ed_hbm.at[wid, 2 * jj + 2], idxA, sem_i)

        pltpu.make_async_copy(x_hbm.at[idxB.at[0]], rowsB, sem_gb).wait()
        pltpu.sync_copy(rowsB, agg_s.at[idxB.at[1]], add=True)

        @pl.when(not_last)
        def _():
            pltpu.make_async_copy(ed_hbm.at[wid, 2 * jj + 2], idxA,
                                  sem_i).wait()
            pltpu.async_copy(x_hbm.at[idxA.at[0]], rowsA, sem_ga)
            pltpu.sync_copy(ed_hbm.at[wid, 2 * jj + 3], idxB)

        return carry

    lax.fori_loop(0, HALF, body, 0)
    plsc.subcore_barrier()

    pltpu.sync_copy(agg_s.at[pl.ds(s * RPT, RPT)],
                    out_hbm.at[c, pl.ds(s * RPT, RPT)])


BLK = 1000
NBLK = N // BLK


def _mlp_pool_body(x_ref, a0_ref, a1_ref, b_ref, Wa_ref, sc_ref, sh_ref,
                   Wb_ref, bb_ref, h_ref, p_ref):
    i = pl.program_id(0)
    hin = x_ref[...] + a0_ref[...] + a1_ref[...]
    t = lax.dot_general(hin, Wa_ref[...], (((1,), (1,)), ((), ())),
                        preferred_element_type=jnp.float32)
    t = jnp.maximum(t * sc_ref[...] + sh_ref[...], 0.0)
    h = lax.dot_general(t, Wb_ref[...], (((1,), (1,)), ((), ())),
                        preferred_element_type=jnp.float32)
    h = jnp.maximum(h + bb_ref[...], 0.0)
    h_ref[...] = h

    @pl.when(i == 0)
    def _():
        p_ref[...] = jnp.zeros_like(p_ref)

    seg = b_ref[0, 0, :][None, :]
    mask = (lax.broadcasted_iota(jnp.int32, (G, BLK), 0) == seg
            ).astype(jnp.float32)
    p_ref[...] += lax.dot_general(mask, h, (((1,), (0,)), ((), ())),
                                  preferred_element_type=jnp.float32)


_mlp_pool = pl.pallas_call(
    _mlp_pool_body,
    grid=(NBLK,),
    in_specs=[
        pl.BlockSpec((BLK, D), lambda i: (i, 0)),
        pl.BlockSpec((BLK, D), lambda i: (i, 0)),
        pl.BlockSpec((BLK, D), lambda i: (i, 0)),
        pl.BlockSpec((1, 1, BLK), lambda i: (i, 0, 0)),
        pl.BlockSpec((H, D), lambda i: (0, 0)),
        pl.BlockSpec((1, H), lambda i: (0, 0)),
        pl.BlockSpec((1, H), lambda i: (0, 0)),
        pl.BlockSpec((H, H), lambda i: (0, 0)),
        pl.BlockSpec((1, H), lambda i: (0, 0)),
    ],
    out_specs=[
        pl.BlockSpec((BLK, H), lambda i: (i, 0)),
        pl.BlockSpec((G, H), lambda i: (0, 0)),
    ],
    out_shape=[
        jax.ShapeDtypeStruct((N, H), jnp.float32),
        jax.ShapeDtypeStruct((G, H), jnp.float32),
    ],
    compiler_params=pltpu.CompilerParams(
        dimension_semantics=("arbitrary",)),
)


def _head_body(p1_ref, p2_ref, p3_ref, W4_ref, b4_ref, W5_ref, b5_ref,
               o_ref, sm_ref):
    p = jnp.concatenate([p1_ref[...], p2_ref[...], p3_ref[...]], axis=1)
    t = lax.dot_general(p, W4_ref[...], (((1,), (1,)), ((), ())),
                        preferred_element_type=jnp.float32)
    t = jnp.maximum(t + b4_ref[...], 0.0)
    o = lax.dot_general(t, W5_ref[...], (((1,), (1,)), ((), ())),
                        preferred_element_type=jnp.float32)
    o = o + b5_ref[...]
    o_ref[...] = o
    m = jnp.max(o, axis=1, keepdims=True)
    e = jnp.exp(o - m)
    sm_ref[...] = e / jnp.sum(e, axis=1, keepdims=True)


_head = pl.pallas_call(
    _head_body,
    out_shape=[
        jax.ShapeDtypeStruct((G, 2), jnp.float32),
        jax.ShapeDtypeStruct((G, 2), jnp.float32),
    ],
)


def kernel(x, edge_index, batch,
           Wa1, ba1, g1, be1, Wb1, bb1,
           Wa2, ba2, g2, be2, Wb2, bb2,
           Wa3, ba3, g3, be3, Wb3, bb3,
           W4, b4, W5, b5):
    pad = EPW_PAD - EPW
    src = jnp.concatenate(
        [edge_index[0].reshape(NW, EPW),
         jnp.zeros((NW, pad), jnp.int32)], axis=1).reshape(NW, ITERS, CHUNK)
    dst = jnp.concatenate(
        [edge_index[1].reshape(NW, EPW),
         jnp.full((NW, pad), N, jnp.int32)], axis=1).reshape(NW, ITERS, CHUNK)
    ed = jnp.stack([src, dst], axis=2)
    z = jnp.zeros((NP, D), jnp.float32)
    batch3 = batch.reshape(NBLK, 1, BLK)

    inv = 1.0 / jnp.sqrt(jnp.float32(1.0 + BN_EPS))
    h = x
    pools = []
    for (Wa, ba, g, be, Wb, bb) in (
            (Wa1, ba1, g1, be1, Wb1, bb1),
            (Wa2, ba2, g2, be2, Wb2, bb2),
            (Wa3, ba3, g3, be3, Wb3, bb3)):
        agg = _sc_aggregate(h, ed, z)
        a0 = agg[0, :N]
        a1 = agg[1, :N]
        scale = (g * inv).reshape(1, H)
        shift = (ba * g * inv + be).reshape(1, H)
        h, p = _mlp_pool(h, a0, a1, batch3, Wa, scale, shift,
                         Wb, bb.reshape(1, H))
        pools.append(p)

    o, sm = _head(pools[0], pools[1], pools[2], W4, b4.reshape(1, 3 * H),
                  W5, b5.reshape(1, 2))
    return (o, sm)


# speedup vs baseline: 3.4123x; 1.0630x over previous
import functools

import jax
import jax.numpy as jnp
from jax import lax
from jax.experimental import pallas as pl
from jax.experimental.pallas import tpu as pltpu
from jax.experimental.pallas import tpu_sc as plsc

N = 10000
E = 320000
D = 128
H = 128
G = 64
BN_EPS = 1e-5

NC = 2
NS = 16
NW = NC * NS
EPW = E // NW
CHUNK = 128
EPW_PAD = 10240
ITERS = EPW_PAD // CHUNK
QITERS = ITERS // 4
NP = 10240
RPT = NP // NS

_sc_mesh = plsc.VectorSubcoreMesh(core_axis_name="c", subcore_axis_name="s")


@functools.partial(
    pl.kernel,
    mesh=_sc_mesh,
    out_type=jax.ShapeDtypeStruct((NC, NP, D), jnp.float32),
    scratch_types=[
        pltpu.VMEM((2, CHUNK), jnp.int32),
        pltpu.VMEM((2, CHUNK), jnp.int32),
        pltpu.VMEM((2, CHUNK), jnp.int32),
        pltpu.VMEM((2, CHUNK), jnp.int32),
        pltpu.VMEM((CHUNK, D), jnp.float32),
        pltpu.VMEM((CHUNK, D), jnp.float32),
        pltpu.VMEM_SHARED((NP, D), jnp.float32),
        pltpu.SemaphoreType.DMA,
        pltpu.SemaphoreType.DMA,
        pltpu.SemaphoreType.DMA,
        pltpu.SemaphoreType.DMA,
    ],
)
def _sc_aggregate(x_hbm, ed_hbm, z_hbm, out_hbm,
                  iA0, iA1, iB0, iB1, rowsA, rowsB, agg_s,
                  sem_ga, sem_gb, sem_ia, sem_ib):
    c = lax.axis_index("c")
    s = lax.axis_index("s")
    wid = s * NC + c

    pltpu.sync_copy(z_hbm.at[pl.ds(s * RPT, RPT)],
                    agg_s.at[pl.ds(s * RPT, RPT)])
    plsc.subcore_barrier()

    pltpu.sync_copy(ed_hbm.at[wid, 0], iA0)
    pltpu.sync_copy(ed_hbm.at[wid, 1], iB0)
    pltpu.sync_copy(ed_hbm.at[wid, 2], iA1)
    pltpu.sync_copy(ed_hbm.at[wid, 3], iB1)
    pltpu.async_copy(x_hbm.at[iA0.at[0]], rowsA, sem_ga)
    pltpu.async_copy(x_hbm.at[iB0.at[0]], rowsB, sem_gb)

    def body(k, carry):
        base = 4 * k
        not_last = k < QITERS - 1

        pltpu.make_async_copy(x_hbm.at[iA0.at[0]], rowsA, sem_ga).wait()
        pltpu.sync_copy(rowsA, agg_s.at[iA0.at[1]], add=True)
        pltpu.async_copy(x_hbm.at[iA1.at[0]], rowsA, sem_ga)

        @pl.when(not_last)
        def _():
            pltpu.async_copy(ed_hbm.at[wid, base + 4], iA0, sem_ia)

        pltpu.make_async_copy(x_hbm.at[iB0.at[0]], rowsB, sem_gb).wait()
        pltpu.sync_copy(rowsB, agg_s.at[iB0.at[1]], add=True)
        pltpu.async_copy(x_hbm.at[iB1.at[0]], rowsB, sem_gb)

        @pl.when(not_last)
        def _():
            pltpu.async_copy(ed_hbm.at[wid, base + 5], iB0, sem_ib)

        pltpu.make_async_copy(x_hbm.at[iA1.at[0]], rowsA, sem_ga).wait()
        pltpu.sync_copy(rowsA, agg_s.at[iA1.at[1]], add=True)

        @pl.when(not_last)
        def _():
            pltpu.make_async_copy(ed_hbm.at[wid, base + 4], iA0,
                                  sem_ia).wait()
            pltpu.async_copy(x_hbm.at[iA0.at[0]], rowsA, sem_ga)
            pltpu.async_copy(ed_hbm.at[wid, base + 6], iA1, sem_ia)

        pltpu.make_async_copy(x_hbm.at[iB1.at[0]], rowsB, sem_gb).wait()
        pltpu.sync_copy(rowsB, agg_s.at[iB1.at[1]], add=True)

        @pl.when(not_last)
        def _():
            pltpu.make_async_copy(ed_hbm.at[wid, base + 5], iB0,
                                  sem_ib).wait()
            pltpu.async_copy(x_hbm.at[iB0.at[0]], rowsB, sem_gb)
            pltpu.async_copy(ed_hbm.at[wid, base + 7], iB1, sem_ib)
            pltpu.make_async_copy(ed_hbm.at[wid, base + 6], iA1,
                                  sem_ia).wait()
            pltpu.make_async_copy(ed_hbm.at[wid, base + 7], iB1,
                                  sem_ib).wait()

        return carry

    lax.fori_loop(0, QITERS, body, 0)
    plsc.subcore_barrier()

    pltpu.sync_copy(agg_s.at[pl.ds(s * RPT, RPT)],
                    out_hbm.at[c, pl.ds(s * RPT, RPT)])


BLK = 1000
NBLK = N // BLK


def _mlp_pool_body(x_ref, a0_ref, a1_ref, b_ref, Wa_ref, sc_ref, sh_ref,
                   Wb_ref, bb_ref, h_ref, p_ref):
    i = pl.program_id(0)
    hin = x_ref[...] + a0_ref[...] + a1_ref[...]
    t = lax.dot_general(hin, Wa_ref[...], (((1,), (1,)), ((), ())),
                        preferred_element_type=jnp.float32)
    t = jnp.maximum(t * sc_ref[...] + sh_ref[...], 0.0)
    h = lax.dot_general(t, Wb_ref[...], (((1,), (1,)), ((), ())),
                        preferred_element_type=jnp.float32)
    h = jnp.maximum(h + bb_ref[...], 0.0)
    h_ref[...] = h

    @pl.when(i == 0)
    def _():
        p_ref[...] = jnp.zeros_like(p_ref)

    seg = b_ref[0, 0, :][None, :]
    mask = (lax.broadcasted_iota(jnp.int32, (G, BLK), 0) == seg
            ).astype(jnp.float32)
    p_ref[...] += lax.dot_general(mask, h, (((1,), (0,)), ((), ())),
                                  preferred_element_type=jnp.float32)


_mlp_pool = pl.pallas_call(
    _mlp_pool_body,
    grid=(NBLK,),
    in_specs=[
        pl.BlockSpec((BLK, D), lambda i: (i, 0)),
        pl.BlockSpec((BLK, D), lambda i: (i, 0)),
        pl.BlockSpec((BLK, D), lambda i: (i, 0)),
        pl.BlockSpec((1, 1, BLK), lambda i: (i, 0, 0)),
        pl.BlockSpec((H, D), lambda i: (0, 0)),
        pl.BlockSpec((1, H), lambda i: (0, 0)),
        pl.BlockSpec((1, H), lambda i: (0, 0)),
        pl.BlockSpec((H, H), lambda i: (0, 0)),
        pl.BlockSpec((1, H), lambda i: (0, 0)),
    ],
    out_specs=[
        pl.BlockSpec((BLK, H), lambda i: (i, 0)),
        pl.BlockSpec((G, H), lambda i: (0, 0)),
    ],
    out_shape=[
        jax.ShapeDtypeStruct((N, H), jnp.float32),
        jax.ShapeDtypeStruct((G, H), jnp.float32),
    ],
    compiler_params=pltpu.CompilerParams(
        dimension_semantics=("arbitrary",)),
)


def _head_body(p1_ref, p2_ref, p3_ref, W4_ref, b4_ref, W5_ref, b5_ref,
               o_ref, sm_ref):
    p = jnp.concatenate([p1_ref[...], p2_ref[...], p3_ref[...]], axis=1)
    t = lax.dot_general(p, W4_ref[...], (((1,), (1,)), ((), ())),
                        preferred_element_type=jnp.float32)
    t = jnp.maximum(t + b4_ref[...], 0.0)
    o = lax.dot_general(t, W5_ref[...], (((1,), (1,)), ((), ())),
                        preferred_element_type=jnp.float32)
    o = o + b5_ref[...]
    o_ref[...] = o
    m = jnp.max(o, axis=1, keepdims=True)
    e = jnp.exp(o - m)
    sm_ref[...] = e / jnp.sum(e, axis=1, keepdims=True)


_head = pl.pallas_call(
    _head_body,
    out_shape=[
        jax.ShapeDtypeStruct((G, 2), jnp.float32),
        jax.ShapeDtypeStruct((G, 2), jnp.float32),
    ],
)


def kernel(x, edge_index, batch,
           Wa1, ba1, g1, be1, Wb1, bb1,
           Wa2, ba2, g2, be2, Wb2, bb2,
           Wa3, ba3, g3, be3, Wb3, bb3,
           W4, b4, W5, b5):
    pad = EPW_PAD - EPW
    src = jnp.concatenate(
        [edge_index[0].reshape(NW, EPW),
         jnp.zeros((NW, pad), jnp.int32)], axis=1).reshape(NW, ITERS, CHUNK)
    dst = jnp.concatenate(
        [edge_index[1].reshape(NW, EPW),
         jnp.full((NW, pad), N, jnp.int32)], axis=1).reshape(NW, ITERS, CHUNK)
    ed = jnp.stack([src, dst], axis=2)
    z = jnp.zeros((NP, D), jnp.float32)
    batch3 = batch.reshape(NBLK, 1, BLK)

    inv = 1.0 / jnp.sqrt(jnp.float32(1.0 + BN_EPS))
    h = x
    pools = []
    for (Wa, ba, g, be, Wb, bb) in (
            (Wa1, ba1, g1, be1, Wb1, bb1),
            (Wa2, ba2, g2, be2, Wb2, bb2),
            (Wa3, ba3, g3, be3, Wb3, bb3)):
        agg = _sc_aggregate(h, ed, z)
        a0 = agg[0, :N]
        a1 = agg[1, :N]
        scale = (g * inv).reshape(1, H)
        shift = (ba * g * inv + be).reshape(1, H)
        h, p = _mlp_pool(h, a0, a1, batch3, Wa, scale, shift,
                         Wb, bb.reshape(1, H))
        pools.append(p)

    o, sm = _head(pools[0], pools[1], pools[2], W4, b4.reshape(1, 3 * H),
                  W5, b5.reshape(1, 2))
    return (o, sm)


# speedup vs baseline: 3.4166x; 1.0012x over previous
import functools

import jax
import jax.numpy as jnp
from jax import lax
from jax.experimental import pallas as pl
from jax.experimental.pallas import tpu as pltpu
from jax.experimental.pallas import tpu_sc as plsc

N = 10000
E = 320000
D = 128
H = 128
G = 64
BN_EPS = 1e-5

NC = 2
NS = 16
NW = NC * NS
EPW = E // NW
CHUNK = 128
EPW_PAD = 10240
ITERS = EPW_PAD // CHUNK
QITERS = ITERS // 4
NP = 10240
RPT = NP // NS

_sc_mesh = plsc.VectorSubcoreMesh(core_axis_name="c", subcore_axis_name="s")


BQ = 16
NBATCH = ITERS // BQ


@functools.partial(
    pl.kernel,
    mesh=_sc_mesh,
    out_type=jax.ShapeDtypeStruct((NC, NP, D), jnp.float32),
    scratch_types=[
        pltpu.VMEM((BQ, 2, CHUNK), jnp.int32),
        pltpu.VMEM((BQ, 2, CHUNK), jnp.int32),
        pltpu.VMEM((CHUNK, D), jnp.float32),
        pltpu.VMEM((CHUNK, D), jnp.float32),
        pltpu.VMEM_SHARED((NP, D), jnp.float32),
        pltpu.SemaphoreType.DMA,
        pltpu.SemaphoreType.DMA,
        pltpu.SemaphoreType.DMA,
        pltpu.SemaphoreType.DMA,
    ],
)
def _sc_aggregate(x_hbm, ed_hbm, z_hbm, out_hbm,
                  ibat0, ibat1, rowsA, rowsB, agg_s,
                  sem_ga, sem_gb, sem_i0, sem_i1):
    c = lax.axis_index("c")
    s = lax.axis_index("s")
    wid = s * NC + c
    ibats = (ibat0, ibat1)
    isems = (sem_i0, sem_i1)

    pltpu.sync_copy(z_hbm.at[pl.ds(s * RPT, RPT)],
                    agg_s.at[pl.ds(s * RPT, RPT)])
    plsc.subcore_barrier()

    pltpu.sync_copy(ed_hbm.at[wid, pl.ds(0, BQ)], ibat0)
    pltpu.async_copy(ed_hbm.at[wid, pl.ds(BQ, BQ)], ibat1, sem_i1)
    pltpu.async_copy(x_hbm.at[ibat0.at[0, 0]], rowsA, sem_ga)
    pltpu.async_copy(x_hbm.at[ibat0.at[1, 0]], rowsB, sem_gb)

    for ch in range(ITERS):
        rows, sem = (rowsA, sem_ga) if ch % 2 == 0 else (rowsB, sem_gb)
        ib = ibats[(ch // BQ) % 2]
        pltpu.make_async_copy(x_hbm.at[ib.at[ch % BQ, 0]], rows, sem).wait()
        pltpu.sync_copy(rows, agg_s.at[ib.at[ch % BQ, 1]], add=True)
        n = ch + 2
        if n < ITERS:
            jb = (n // BQ) % 2
            if n % BQ == 0:
                pltpu.make_async_copy(
                    ed_hbm.at[wid, pl.ds(n, BQ)], ibats[jb],
                    isems[jb]).wait()
            pltpu.async_copy(x_hbm.at[ibats[jb].at[n % BQ, 0]], rows, sem)
        if ch % BQ == BQ - 1:
            k2 = ch // BQ + 2
            if k2 < NBATCH:
                pltpu.async_copy(ed_hbm.at[wid, pl.ds(k2 * BQ, BQ)],
                                 ibats[k2 % 2], isems[k2 % 2])
    plsc.subcore_barrier()

    pltpu.sync_copy(agg_s.at[pl.ds(s * RPT, RPT)],
                    out_hbm.at[c, pl.ds(s * RPT, RPT)])


BLK = 1000
NBLK = N // BLK


def _mlp_pool_body(x_ref, a0_ref, a1_ref, b_ref, Wa_ref, sc_ref, sh_ref,
                   Wb_ref, bb_ref, h_ref, p_ref):
    i = pl.program_id(0)
    hin = x_ref[...] + a0_ref[...] + a1_ref[...]
    t = lax.dot_general(hin, Wa_ref[...], (((1,), (1,)), ((), ())),
                        preferred_element_type=jnp.float32)
    t = jnp.maximum(t * sc_ref[...] + sh_ref[...], 0.0)
    h = lax.dot_general(t, Wb_ref[...], (((1,), (1,)), ((), ())),
                        preferred_element_type=jnp.float32)
    h = jnp.maximum(h + bb_ref[...], 0.0)
    h_ref[...] = h

    @pl.when(i == 0)
    def _():
        p_ref[...] = jnp.zeros_like(p_ref)

    seg = b_ref[0, 0, :][None, :]
    mask = (lax.broadcasted_iota(jnp.int32, (G, BLK), 0) == seg
            ).astype(jnp.float32)
    p_ref[...] += lax.dot_general(mask, h, (((1,), (0,)), ((), ())),
                                  preferred_element_type=jnp.float32)


_mlp_pool = pl.pallas_call(
    _mlp_pool_body,
    grid=(NBLK,),
    in_specs=[
        pl.BlockSpec((BLK, D), lambda i: (i, 0)),
        pl.BlockSpec((BLK, D), lambda i: (i, 0)),
        pl.BlockSpec((BLK, D), lambda i: (i, 0)),
        pl.BlockSpec((1, 1, BLK), lambda i: (i, 0, 0)),
        pl.BlockSpec((H, D), lambda i: (0, 0)),
        pl.BlockSpec((1, H), lambda i: (0, 0)),
        pl.BlockSpec((1, H), lambda i: (0, 0)),
        pl.BlockSpec((H, H), lambda i: (0, 0)),
        pl.BlockSpec((1, H), lambda i: (0, 0)),
    ],
    out_specs=[
        pl.BlockSpec((BLK, H), lambda i: (i, 0)),
        pl.BlockSpec((G, H), lambda i: (0, 0)),
    ],
    out_shape=[
        jax.ShapeDtypeStruct((N, H), jnp.float32),
        jax.ShapeDtypeStruct((G, H), jnp.float32),
    ],
    compiler_params=pltpu.CompilerParams(
        dimension_semantics=("arbitrary",)),
)


def _head_body(p1_ref, p2_ref, p3_ref, W4_ref, b4_ref, W5_ref, b5_ref,
               o_ref, sm_ref):
    p = jnp.concatenate([p1_ref[...], p2_ref[...], p3_ref[...]], axis=1)
    t = lax.dot_general(p, W4_ref[...], (((1,), (1,)), ((), ())),
                        preferred_element_type=jnp.float32)
    t = jnp.maximum(t + b4_ref[...], 0.0)
    o = lax.dot_general(t, W5_ref[...], (((1,), (1,)), ((), ())),
                        preferred_element_type=jnp.float32)
    o = o + b5_ref[...]
    o_ref[...] = o
    m = jnp.max(o, axis=1, keepdims=True)
    e = jnp.exp(o - m)
    sm_ref[...] = e / jnp.sum(e, axis=1, keepdims=True)


_head = pl.pallas_call(
    _head_body,
    out_shape=[
        jax.ShapeDtypeStruct((G, 2), jnp.float32),
        jax.ShapeDtypeStruct((G, 2), jnp.float32),
    ],
)


def kernel(x, edge_index, batch,
           Wa1, ba1, g1, be1, Wb1, bb1,
           Wa2, ba2, g2, be2, Wb2, bb2,
           Wa3, ba3, g3, be3, Wb3, bb3,
           W4, b4, W5, b5):
    pad = EPW_PAD - EPW
    src = jnp.concatenate(
        [edge_index[0].reshape(NW, EPW),
         jnp.zeros((NW, pad), jnp.int32)], axis=1).reshape(NW, ITERS, CHUNK)
    dst = jnp.concatenate(
        [edge_index[1].reshape(NW, EPW),
         jnp.full((NW, pad), N, jnp.int32)], axis=1).reshape(NW, ITERS, CHUNK)
    ed = jnp.stack([src, dst], axis=2)
    z = jnp.zeros((NP, D), jnp.float32)
    batch3 = batch.reshape(NBLK, 1, BLK)

    inv = 1.0 / jnp.sqrt(jnp.float32(1.0 + BN_EPS))
    h = x
    pools = []
    for (Wa, ba, g, be, Wb, bb) in (
            (Wa1, ba1, g1, be1, Wb1, bb1),
            (Wa2, ba2, g2, be2, Wb2, bb2),
            (Wa3, ba3, g3, be3, Wb3, bb3)):
        agg = _sc_aggregate(h, ed, z)
        a0 = agg[0, :N]
        a1 = agg[1, :N]
        scale = (g * inv).reshape(1, H)
        shift = (ba * g * inv + be).reshape(1, H)
        h, p = _mlp_pool(h, a0, a1, batch3, Wa, scale, shift,
                         Wb, bb.reshape(1, H))
        pools.append(p)

    o, sm = _head(pools[0], pools[1], pools[2], W4, b4.reshape(1, 3 * H),
                  W5, b5.reshape(1, 2))
    return (o, sm)
